# baseline (speedup 1.0000x reference)
import functools

import jax
import jax.numpy as jnp
from jax import lax
from jax.experimental import pallas as pl
from jax.experimental.pallas import tpu as pltpu
from jax.experimental.pallas import tpu_sc as plsc

N = 10000
NPAD = 10112
D = 128
E = 320000
K = 288
NTILES = 32
CHT0 = 41
CHT1 = 29
CHTM = max(CHT0, CHT1)
EPAD = 16 * (CHT0 + CHT1) * K
NG = 128
GB = 136
POOL_T = NPAD // NTILES
BPAD = 336

_mesh = plsc.VectorSubcoreMesh(core_axis_name="c", subcore_axis_name="s")
_sc_params = pltpu.CompilerParams(needs_layout_passes=False,
                                 use_tc_tiling_on_sc=False)


@functools.partial(
    pl.kernel, mesh=_mesh,
    out_type=jax.ShapeDtypeStruct((NTILES, NPAD), jnp.float32),
    scratch_types=[pltpu.VMEM((CHTM * K,), jnp.int32),
                   pltpu.VMEM((NPAD,), jnp.float32)],
    compiler_params=_sc_params,
)
def _deg_kernel(pk_hbm, out_hbm, idxv, accv):
    wid = lax.axis_index("s") * 2 + lax.axis_index("c")
    pltpu.sync_copy(pk_hbm.at[wid], idxv)
    zero16 = jnp.zeros((16,), jnp.float32)

    def _z(i, c):
        accv[pl.ds(i * 16, 16)] = zero16
        return c
    lax.fori_loop(0, NPAD // 16, _z, 0)

    ones16 = jnp.ones((16,), jnp.float32)

    def _s(i, c):
        d16 = lax.shift_right_logical(idxv[pl.ds(i * 16, 16)], 16)
        plsc.addupdate_scatter(accv, [d16], ones16)
        return c
    lax.fori_loop(0, CHTM * K // 16, _s, 0)
    pltpu.sync_copy(accv, out_hbm.at[wid])


@functools.partial(
    pl.kernel, mesh=_mesh,
    out_type=jax.ShapeDtypeStruct((2, NPAD, D), jnp.float32),
    scratch_types=[pltpu.VMEM((CHTM * K,), jnp.int32),
                   pltpu.VMEM((K,), jnp.int32),
                   pltpu.VMEM((K,), jnp.int32),
                   pltpu.VMEM((K, D), jnp.float32),
                   pltpu.VMEM_SHARED((NPAD, D), jnp.float32),
                   pltpu.SemaphoreType.DMA],
    compiler_params=_sc_params,
)
def _conv_kernel(xs_hbm, pk_hbm, zeros_hbm, out_hbm,
                 pkv, srcx, dstx, buf, acc, sem):
    cid = lax.axis_index("c")
    sid = lax.axis_index("s")
    wid = sid * 2 + cid
    pltpu.sync_copy(pk_hbm.at[wid], pkv)
    nrows = NPAD // 16
    r0 = sid * nrows
    pltpu.sync_copy(zeros_hbm.at[pl.ds(r0, nrows)], acc.at[pl.ds(r0, nrows)])
    plsc.subcore_barrier()

    ncht = jnp.where(cid == 0, CHT0, CHT1)

    def _body(j, c):
        base = j * K
        for i in range(K // 16):
            p = pkv[pl.ds(base + i * 16, 16)]
            srcx[pl.ds(i * 16, 16)] = p & 0xFFFF
            dstx[pl.ds(i * 16, 16)] = lax.shift_right_logical(p, 16)
        pltpu.async_copy(xs_hbm.at[srcx], buf, sem).wait()
        pltpu.sync_copy(buf, acc.at[dstx], add=True)
        return c
    lax.fori_loop(0, ncht, _body, 0)
    plsc.subcore_barrier()
    pltpu.sync_copy(acc.at[pl.ds(r0, nrows)],
                    out_hbm.at[cid, pl.ds(r0, nrows)])


@functools.partial(
    pl.kernel, mesh=_mesh,
    out_type=[jax.ShapeDtypeStruct((NTILES, GB * D), jnp.float32),
              jax.ShapeDtypeStruct((NTILES, GB * D), jnp.float32)],
    scratch_types=[pltpu.VMEM((POOL_T * D,), jnp.float32),
                   pltpu.VMEM((BPAD,), jnp.int32),
                   pltpu.VMEM((GB * D,), jnp.float32),
                   pltpu.VMEM((GB * D,), jnp.float32)],
    compiler_params=_sc_params,
)
def _pool_kernel(x2_hbm, batch_hbm, outmax_hbm, outsum_hbm, xv, bv, mxb, smb):
    wid = lax.axis_index("s") * 2 + lax.axis_index("c")
    pltpu.sync_copy(x2_hbm.at[wid], xv)
    pltpu.sync_copy(batch_hbm.at[wid], bv)
    neg16 = jnp.full((16,), -jnp.inf, jnp.float32)
    zero16 = jnp.zeros((16,), jnp.float32)

    def _z(i, c):
        mxb[pl.ds(i * 16, 16)] = neg16
        smb[pl.ds(i * 16, 16)] = zero16
        return c
    lax.fori_loop(0, GB * D // 16, _z, 0)

    def _n(i, c):
        g = bv[pl.ds(i, 16)][0]
        off = g * D
        xoff = i * D
        for j in range(D // 16):
            v = xv[pl.ds(xoff + j * 16, 16)]
            mxb[pl.ds(off + j * 16, 16)] = jnp.maximum(
                mxb[pl.ds(off + j * 16, 16)], v)
            smb[pl.ds(off + j * 16, 16)] = smb[pl.ds(off + j * 16, 16)] + v
        return c
    lax.fori_loop(0, POOL_T, _n, 0)
    pltpu.sync_copy(mxb, outmax_hbm.at[wid])
    pltpu.sync_copy(smb, outsum_hbm.at[wid])


def _prep_body(degp_ref, x_ref, dinv_ref, xs_ref):
    deg = jnp.sum(degp_ref[...], axis=1, keepdims=True) + 1.0
    dinv = lax.rsqrt(deg)
    mask = lax.broadcasted_iota(jnp.int32, (NPAD, 1), 0) < N
    dinv = jnp.where(mask, dinv, 0.0)
    dinv_ref[...] = dinv
    xs_ref[...] = x_ref[...] * dinv


_prep = pl.pallas_call(
    _prep_body,
    out_shape=[jax.ShapeDtypeStruct((NPAD, 1), jnp.float32),
               jax.ShapeDtypeStruct((NPAD, D), jnp.float32)],
)


def _tail_body(p_ref, dinv_ref, xin_ref, w_ref, b_ref, xn_ref, xsn_ref):
    t = p_ref[0] + p_ref[1]
    dinv = dinv_ref[...]
    agg = dinv * t + (dinv * dinv) * xin_ref[...]
    h = jnp.maximum(
        jnp.dot(agg, w_ref[...], preferred_element_type=jnp.float32)
        + b_ref[...], 0.0)
    xn = h + xin_ref[...]
    xn_ref[...] = xn
    xsn_ref[...] = xn * dinv


_tail = pl.pallas_call(
    _tail_body,
    out_shape=[jax.ShapeDtypeStruct((NPAD, D), jnp.float32),
               jax.ShapeDtypeStruct((NPAD, D), jnp.float32)],
)


def _head_body(mx_ref, sm_ref, mol_ref, wm_ref, bm_ref, g_ref, be_ref,
               wa_ref, ba_ref, wb_ref, bb_ref, out_ref):
    mx = jnp.full((GB, D), -jnp.inf, jnp.float32)
    sm = jnp.zeros((GB, D), jnp.float32)
    for w in range(NTILES):
        mx = jnp.maximum(mx, mx_ref[pl.ds(w * GB, GB), :])
        sm = sm + sm_ref[pl.ds(w * GB, GB), :]
    mx = mx[:NG]
    sm = sm[:NG]
    mol = jnp.dot(mol_ref[...], wm_ref[...],
                  preferred_element_type=jnp.float32) + bm_ref[...]
    mol = jnp.maximum(g_ref[...] * mol + be_ref[...], 0.0)
    cat = jnp.concatenate([mx, sm, mol], axis=1)
    hid = jnp.maximum(
        jnp.dot(cat, wa_ref[...], preferred_element_type=jnp.float32)
        + ba_ref[...], 0.0)
    out_ref[...] = jnp.dot(hid, wb_ref[...],
                           preferred_element_type=jnp.float32) + bb_ref[...]


_headk = pl.pallas_call(
    _head_body,
    out_shape=jax.ShapeDtypeStruct((NG, 1), jnp.float32),
)


def kernel(x, edge_index, batch, molecule_features, W1, b1, W2, b2,
           Wm, bm, gamma, beta, Wa, ba, Wb, bb):
    src = edge_index[0]
    dst = edge_index[1]
    padi = jnp.full((EPAD - E,), N, jnp.int32)
    n0 = 16 * CHT0 * K

    def _split(flat):
        fill = N | (N << 16)
        p0 = jnp.pad(flat[:n0].reshape(16, CHT0, K),
                     ((0, 0), (0, CHTM - CHT0), (0, 0)), constant_values=fill)
        p1 = jnp.pad(flat[n0:].reshape(16, CHT1, K),
                     ((0, 0), (0, CHTM - CHT1), (0, 0)), constant_values=fill)
        return jnp.stack([p0, p1], axis=1).reshape(NTILES, CHTM, K)

    pk = (jnp.concatenate([src, padi])
          | (jnp.concatenate([dst, padi]) << 16))
    pk3 = _split(pk)
    pk2 = pk3.reshape(NTILES, CHTM * K)
    xpad = jnp.pad(x, ((0, NPAD - N), (0, 0)))
    zeros = jnp.zeros((NPAD, D), jnp.float32)

    degp = _deg_kernel(pk2)
    dinv, xs = _prep(degp.T, xpad)
    p1 = _conv_kernel(xs, pk2, zeros)
    x1, xs1 = _tail(p1, dinv, xpad, W1, b1.reshape(1, D))
    p2 = _conv_kernel(xs1, pk2, zeros)
    x2, _ = _tail(p2, dinv, x1, W2, b2.reshape(1, D))

    bp = jnp.full((NTILES, BPAD), NG, jnp.int32)
    bp = bp.at[:, :POOL_T].set(
        jnp.pad(batch, (0, NPAD - N), constant_values=NG).reshape(
            NTILES, POOL_T))
    mxp, smp = _pool_kernel(x2.reshape(NTILES, POOL_T * D), bp)
    out = _headk(mxp.reshape(NTILES * GB, D), smp.reshape(NTILES * GB, D),
                 molecule_features, Wm, bm.reshape(1, -1),
                 gamma.reshape(1, -1), beta.reshape(1, -1),
                 Wa, ba.reshape(1, D), Wb, bb.reshape(1, 1))
    return out


# speedup vs baseline: 1.0217x; 1.0217x over previous
import functools

import jax
import jax.numpy as jnp
from jax import lax
from jax.experimental import pallas as pl
from jax.experimental.pallas import tpu as pltpu
from jax.experimental.pallas import tpu_sc as plsc

N = 10000
NPAD = 10112
D = 128
E = 320000
K = 192
NTILES = 32
CHT0 = 61
CHT1 = 44
CHTM = max(CHT0, CHT1)
EPAD = 16 * (CHT0 + CHT1) * K
NG = 128
GB = 136
POOL_T = NPAD // NTILES
BPAD = 336

_mesh = plsc.VectorSubcoreMesh(core_axis_name="c", subcore_axis_name="s")
_sc_params = pltpu.CompilerParams(needs_layout_passes=False,
                                 use_tc_tiling_on_sc=False)


@functools.partial(
    pl.kernel, mesh=_mesh,
    out_type=jax.ShapeDtypeStruct((NTILES, NPAD), jnp.float32),
    scratch_types=[pltpu.VMEM((CHTM * K,), jnp.int32),
                   pltpu.VMEM((NPAD,), jnp.float32)],
    compiler_params=_sc_params,
)
def _deg_kernel(dst_hbm, out_hbm, idxv, accv):
    wid = lax.axis_index("s") * 2 + lax.axis_index("c")
    pltpu.sync_copy(dst_hbm.at[wid], idxv)
    zero16 = jnp.zeros((16,), jnp.float32)

    def _z(i, c):
        accv[pl.ds(i * 16, 16)] = zero16
        return c
    lax.fori_loop(0, NPAD // 16, _z, 0)

    ones16 = jnp.ones((16,), jnp.float32)

    def _s(i, c):
        plsc.addupdate_scatter(accv, [idxv[pl.ds(i * 16, 16)]], ones16)
        return c
    lax.fori_loop(0, CHTM * K // 16, _s, 0)
    pltpu.sync_copy(accv, out_hbm.at[wid])


@functools.partial(
    pl.kernel, mesh=_mesh,
    out_type=jax.ShapeDtypeStruct((2, NPAD, D), jnp.float32),
    scratch_types=[pltpu.VMEM((CHTM, K), jnp.int32),
                   pltpu.VMEM((CHTM, K), jnp.int32),
                   pltpu.VMEM((K, D), jnp.float32),
                   pltpu.VMEM_SHARED((NPAD, D), jnp.float32),
                   pltpu.SemaphoreType.DMA],
    compiler_params=_sc_params,
)
def _conv_kernel(xs_hbm, src_hbm, dst_hbm, zeros_hbm, out_hbm,
                 srcv, dstv, buf, acc, sem):
    cid = lax.axis_index("c")
    sid = lax.axis_index("s")
    wid = sid * 2 + cid
    pltpu.sync_copy(src_hbm.at[wid], srcv)
    pltpu.sync_copy(dst_hbm.at[wid], dstv)
    nrows = NPAD // 16
    r0 = sid * nrows
    pltpu.sync_copy(zeros_hbm.at[pl.ds(r0, nrows)], acc.at[pl.ds(r0, nrows)])
    plsc.subcore_barrier()

    ncht = jnp.where(cid == 0, CHT0, CHT1)

    def _body(j, c):
        pltpu.async_copy(xs_hbm.at[srcv.at[j]], buf, sem).wait()
        pltpu.sync_copy(buf, acc.at[dstv.at[j]], add=True)
        return c
    lax.fori_loop(0, ncht, _body, 0)
    plsc.subcore_barrier()
    pltpu.sync_copy(acc.at[pl.ds(r0, nrows)],
                    out_hbm.at[cid, pl.ds(r0, nrows)])


@functools.partial(
    pl.kernel, mesh=_mesh,
    out_type=[jax.ShapeDtypeStruct((NTILES, GB * D), jnp.float32),
              jax.ShapeDtypeStruct((NTILES, GB * D), jnp.float32)],
    scratch_types=[pltpu.VMEM((POOL_T * D,), jnp.float32),
                   pltpu.VMEM((BPAD,), jnp.int32),
                   pltpu.VMEM((GB * D,), jnp.float32),
                   pltpu.VMEM((GB * D,), jnp.float32)],
    compiler_params=_sc_params,
)
def _pool_kernel(x2_hbm, batch_hbm, outmax_hbm, outsum_hbm, xv, bv, mxb, smb):
    wid = lax.axis_index("s") * 2 + lax.axis_index("c")
    pltpu.sync_copy(x2_hbm.at[wid], xv)
    pltpu.sync_copy(batch_hbm.at[wid], bv)
    neg16 = jnp.full((16,), -jnp.inf, jnp.float32)
    zero16 = jnp.zeros((16,), jnp.float32)

    def _z(i, c):
        mxb[pl.ds(i * 16, 16)] = neg16
        smb[pl.ds(i * 16, 16)] = zero16
        return c
    lax.fori_loop(0, GB * D // 16, _z, 0)

    def _n(i, c):
        g = bv[pl.ds(i, 16)][0]
        off = g * D
        xoff = i * D
        for j in range(D // 16):
            v = xv[pl.ds(xoff + j * 16, 16)]
            mxb[pl.ds(off + j * 16, 16)] = jnp.maximum(
                mxb[pl.ds(off + j * 16, 16)], v)
            smb[pl.ds(off + j * 16, 16)] = smb[pl.ds(off + j * 16, 16)] + v
        return c
    lax.fori_loop(0, POOL_T, _n, 0)
    pltpu.sync_copy(mxb, outmax_hbm.at[wid])
    pltpu.sync_copy(smb, outsum_hbm.at[wid])


def _prep_body(degp_ref, x_ref, dinv_ref, xs_ref):
    deg = jnp.sum(degp_ref[...], axis=1, keepdims=True) + 1.0
    dinv = lax.rsqrt(deg)
    mask = lax.broadcasted_iota(jnp.int32, (NPAD, 1), 0) < N
    dinv = jnp.where(mask, dinv, 0.0)
    dinv_ref[...] = dinv
    xs_ref[...] = x_ref[...] * dinv


_prep = pl.pallas_call(
    _prep_body,
    out_shape=[jax.ShapeDtypeStruct((NPAD, 1), jnp.float32),
               jax.ShapeDtypeStruct((NPAD, D), jnp.float32)],
)


def _tail_body(p_ref, dinv_ref, xin_ref, w_ref, b_ref, xn_ref, xsn_ref):
    t = p_ref[0] + p_ref[1]
    dinv = dinv_ref[...]
    agg = dinv * t + (dinv * dinv) * xin_ref[...]
    h = jnp.maximum(
        jnp.dot(agg, w_ref[...], preferred_element_type=jnp.float32)
        + b_ref[...], 0.0)
    xn = h + xin_ref[...]
    xn_ref[...] = xn
    xsn_ref[...] = xn * dinv


_tail = pl.pallas_call(
    _tail_body,
    out_shape=[jax.ShapeDtypeStruct((NPAD, D), jnp.float32),
               jax.ShapeDtypeStruct((NPAD, D), jnp.float32)],
)


def _head_body(mx_ref, sm_ref, mol_ref, wm_ref, bm_ref, g_ref, be_ref,
               wa_ref, ba_ref, wb_ref, bb_ref, out_ref):
    mx = jnp.full((GB, D), -jnp.inf, jnp.float32)
    sm = jnp.zeros((GB, D), jnp.float32)
    for w in range(NTILES):
        mx = jnp.maximum(mx, mx_ref[pl.ds(w * GB, GB), :])
        sm = sm + sm_ref[pl.ds(w * GB, GB), :]
    mx = mx[:NG]
    sm = sm[:NG]
    mol = jnp.dot(mol_ref[...], wm_ref[...],
                  preferred_element_type=jnp.float32) + bm_ref[...]
    mol = jnp.maximum(g_ref[...] * mol + be_ref[...], 0.0)
    cat = jnp.concatenate([mx, sm, mol], axis=1)
    hid = jnp.maximum(
        jnp.dot(cat, wa_ref[...], preferred_element_type=jnp.float32)
        + ba_ref[...], 0.0)
    out_ref[...] = jnp.dot(hid, wb_ref[...],
                           preferred_element_type=jnp.float32) + bb_ref[...]


_headk = pl.pallas_call(
    _head_body,
    out_shape=jax.ShapeDtypeStruct((NG, 1), jnp.float32),
)


def kernel(x, edge_index, batch, molecule_features, W1, b1, W2, b2,
           Wm, bm, gamma, beta, Wa, ba, Wb, bb):
    src = edge_index[0]
    dst = edge_index[1]
    padi = jnp.full((EPAD - E,), N, jnp.int32)
    n0 = 16 * CHT0 * K

    def _split(flat):
        p0 = jnp.pad(flat[:n0].reshape(16, CHT0, K),
                     ((0, 0), (0, CHTM - CHT0), (0, 0)), constant_values=N)
        p1 = jnp.pad(flat[n0:].reshape(16, CHT1, K),
                     ((0, 0), (0, CHTM - CHT1), (0, 0)), constant_values=N)
        return jnp.stack([p0, p1], axis=1).reshape(NTILES, CHTM, K)

    srcp = _split(jnp.concatenate([src, padi]))
    dst3 = _split(jnp.concatenate([dst, padi]))
    dst2 = dst3.reshape(NTILES, CHTM * K)
    xpad = jnp.pad(x, ((0, NPAD - N), (0, 0)))
    zeros = jnp.zeros((NPAD, D), jnp.float32)

    degp = _deg_kernel(dst2)
    dinv, xs = _prep(degp.T, xpad)
    p1 = _conv_kernel(xs, srcp, dst3, zeros)
    x1, xs1 = _tail(p1, dinv, xpad, W1, b1.reshape(1, D))
    p2 = _conv_kernel(xs1, srcp, dst3, zeros)
    x2, _ = _tail(p2, dinv, x1, W2, b2.reshape(1, D))

    bp = jnp.full((NTILES, BPAD), NG, jnp.int32)
    bp = bp.at[:, :POOL_T].set(
        jnp.pad(batch, (0, NPAD - N), constant_values=NG).reshape(
            NTILES, POOL_T))
    mxp, smp = _pool_kernel(x2.reshape(NTILES, POOL_T * D), bp)
    out = _headk(mxp.reshape(NTILES * GB, D), smp.reshape(NTILES * GB, D),
                 molecule_features, Wm, bm.reshape(1, -1),
                 gamma.reshape(1, -1), beta.reshape(1, -1),
                 Wa, ba.reshape(1, D), Wb, bb.reshape(1, 1))
    return out


# speedup vs baseline: 1.1183x; 1.0946x over previous
import functools

import jax
import jax.numpy as jnp
from jax import lax
from jax.experimental import pallas as pl
from jax.experimental.pallas import tpu as pltpu
from jax.experimental.pallas import tpu_sc as plsc

N = 10000
NPAD = 10112
D = 128
E = 320000
K = 144
NTILES = 32
CHT0 = 86
CHT1 = 54
CHTM = max(CHT0, CHT1)
EPAD = 16 * (CHT0 + CHT1) * K
NG = 128
GB = 136
POOL_T = NPAD // NTILES
BPAD = 336

_mesh = plsc.VectorSubcoreMesh(core_axis_name="c", subcore_axis_name="s")
_sc_params = pltpu.CompilerParams(needs_layout_passes=False,
                                 use_tc_tiling_on_sc=False)


@functools.partial(
    pl.kernel, mesh=_mesh,
    out_type=jax.ShapeDtypeStruct((NTILES, NPAD), jnp.float32),
    scratch_types=[pltpu.VMEM((CHTM * K,), jnp.int32),
                   pltpu.VMEM((NPAD,), jnp.float32)],
    compiler_params=_sc_params,
)
def _deg_kernel(pk_hbm, out_hbm, idxv, accv):
    wid = lax.axis_index("s") * 2 + lax.axis_index("c")
    pltpu.sync_copy(pk_hbm.at[wid], idxv)
    zero16 = jnp.zeros((16,), jnp.float32)

    def _z(i, c):
        accv[pl.ds(i * 16, 16)] = zero16
        return c
    lax.fori_loop(0, NPAD // 16, _z, 0)

    ones16 = jnp.ones((16,), jnp.float32)

    def _s(i, c):
        d16 = lax.shift_right_logical(idxv[pl.ds(i * 16, 16)], 16)
        plsc.addupdate_scatter(accv, [d16], ones16)
        return c
    lax.fori_loop(0, CHTM * K // 16, _s, 0)
    pltpu.sync_copy(accv, out_hbm.at[wid])


@functools.partial(
    pl.kernel, mesh=_mesh,
    out_type=jax.ShapeDtypeStruct((2, NPAD, D), jnp.float32),
    scratch_types=[pltpu.VMEM((CHTM * K,), jnp.int32),
                   pltpu.VMEM((K,), jnp.int32),
                   pltpu.VMEM((K,), jnp.int32),
                   pltpu.VMEM((K,), jnp.int32),
                   pltpu.VMEM((K,), jnp.int32),
                   pltpu.VMEM((K, D), jnp.float32),
                   pltpu.VMEM((K, D), jnp.float32),
                   pltpu.VMEM_SHARED((NPAD, D), jnp.float32),
                   pltpu.SemaphoreType.DMA,
                   pltpu.SemaphoreType.DMA],
    compiler_params=_sc_params,
)
def _conv_kernel(xs_hbm, pk_hbm, zeros_hbm, out_hbm,
                 pkv, srcxa, dstxa, srcxb, dstxb, bufa, bufb, acc,
                 sema, semb):
    cid = lax.axis_index("c")
    sid = lax.axis_index("s")
    wid = sid * 2 + cid
    pltpu.sync_copy(pk_hbm.at[wid], pkv)
    nrows = NPAD // 16
    r0 = sid * nrows
    pltpu.sync_copy(zeros_hbm.at[pl.ds(r0, nrows)], acc.at[pl.ds(r0, nrows)])
    plsc.subcore_barrier()

    ncht = jnp.where(cid == 0, CHT0, CHT1)

    def _unpack(j, sx, dx):
        base = j * K
        for i in range(K // 16):
            p = pkv[pl.ds(base + i * 16, 16)]
            sx[pl.ds(i * 16, 16)] = p & 0xFFFF
            dx[pl.ds(i * 16, 16)] = lax.shift_right_logical(p, 16)

    _unpack(0, srcxa, dstxa)
    pltpu.async_copy(xs_hbm.at[srcxa], bufa, sema)

    def _pair(jj, c):
        j0 = jj * 2
        _unpack(j0 + 1, srcxb, dstxb)
        pltpu.make_async_copy(xs_hbm.at[srcxa], bufa, sema).wait()
        pltpu.async_copy(xs_hbm.at[srcxb], bufb, semb)
        pltpu.sync_copy(bufa, acc.at[dstxa], add=True)

        @pl.when(j0 + 2 < ncht)
        def _():
            _unpack(j0 + 2, srcxa, dstxa)
            pltpu.async_copy(xs_hbm.at[srcxa], bufa, sema)
        pltpu.make_async_copy(xs_hbm.at[srcxb], bufb, semb).wait()
        pltpu.sync_copy(bufb, acc.at[dstxb], add=True)
        return c
    lax.fori_loop(0, ncht // 2, _pair, 0)
    plsc.subcore_barrier()
    pltpu.sync_copy(acc.at[pl.ds(r0, nrows)],
                    out_hbm.at[cid, pl.ds(r0, nrows)])


@functools.partial(
    pl.kernel, mesh=_mesh,
    out_type=[jax.ShapeDtypeStruct((NTILES, GB * D), jnp.float32),
              jax.ShapeDtypeStruct((NTILES, GB * D), jnp.float32)],
    scratch_types=[pltpu.VMEM((POOL_T * D,), jnp.float32),
                   pltpu.VMEM((BPAD,), jnp.int32),
                   pltpu.VMEM((GB * D,), jnp.float32),
                   pltpu.VMEM((GB * D,), jnp.float32)],
    compiler_params=_sc_params,
)
def _pool_kernel(x2_hbm, batch_hbm, outmax_hbm, outsum_hbm, xv, bv, mxb, smb):
    wid = lax.axis_index("s") * 2 + lax.axis_index("c")
    pltpu.sync_copy(x2_hbm.at[wid], xv)
    pltpu.sync_copy(batch_hbm.at[wid], bv)
    neg16 = jnp.full((16,), -jnp.inf, jnp.float32)
    zero16 = jnp.zeros((16,), jnp.float32)

    def _z(i, c):
        mxb[pl.ds(i * 16, 16)] = neg16
        smb[pl.ds(i * 16, 16)] = zero16
        return c
    lax.fori_loop(0, GB * D // 16, _z, 0)

    def _n(i, c):
        g = bv[pl.ds(i, 16)][0]
        off = g * D
        xoff = i * D
        for j in range(D // 16):
            v = xv[pl.ds(xoff + j * 16, 16)]
            mxb[pl.ds(off + j * 16, 16)] = jnp.maximum(
                mxb[pl.ds(off + j * 16, 16)], v)
            smb[pl.ds(off + j * 16, 16)] = smb[pl.ds(off + j * 16, 16)] + v
        return c
    lax.fori_loop(0, POOL_T, _n, 0)
    pltpu.sync_copy(mxb, outmax_hbm.at[wid])
    pltpu.sync_copy(smb, outsum_hbm.at[wid])


def _prep_body(degp_ref, x_ref, dinv_ref, xs_ref):
    deg = jnp.sum(degp_ref[...], axis=1, keepdims=True) + 1.0
    dinv = lax.rsqrt(deg)
    mask = lax.broadcasted_iota(jnp.int32, (NPAD, 1), 0) < N
    dinv = jnp.where(mask, dinv, 0.0)
    dinv_ref[...] = dinv
    xs_ref[...] = x_ref[...] * dinv


_prep = pl.pallas_call(
    _prep_body,
    out_shape=[jax.ShapeDtypeStruct((NPAD, 1), jnp.float32),
               jax.ShapeDtypeStruct((NPAD, D), jnp.float32)],
)


def _tail_body(p_ref, dinv_ref, xin_ref, w_ref, b_ref, xn_ref, xsn_ref):
    t = p_ref[0] + p_ref[1]
    dinv = dinv_ref[...]
    agg = dinv * t + (dinv * dinv) * xin_ref[...]
    h = jnp.maximum(
        jnp.dot(agg, w_ref[...], preferred_element_type=jnp.float32)
        + b_ref[...], 0.0)
    xn = h + xin_ref[...]
    xn_ref[...] = xn
    xsn_ref[...] = xn * dinv


_tail = pl.pallas_call(
    _tail_body,
    out_shape=[jax.ShapeDtypeStruct((NPAD, D), jnp.float32),
               jax.ShapeDtypeStruct((NPAD, D), jnp.float32)],
)


def _head_body(mx_ref, sm_ref, mol_ref, wm_ref, bm_ref, g_ref, be_ref,
               wa_ref, ba_ref, wb_ref, bb_ref, out_ref):
    mx = jnp.full((GB, D), -jnp.inf, jnp.float32)
    sm = jnp.zeros((GB, D), jnp.float32)
    for w in range(NTILES):
        mx = jnp.maximum(mx, mx_ref[pl.ds(w * GB, GB), :])
        sm = sm + sm_ref[pl.ds(w * GB, GB), :]
    mx = mx[:NG]
    sm = sm[:NG]
    mol = jnp.dot(mol_ref[...], wm_ref[...],
                  preferred_element_type=jnp.float32) + bm_ref[...]
    mol = jnp.maximum(g_ref[...] * mol + be_ref[...], 0.0)
    cat = jnp.concatenate([mx, sm, mol], axis=1)
    hid = jnp.maximum(
        jnp.dot(cat, wa_ref[...], preferred_element_type=jnp.float32)
        + ba_ref[...], 0.0)
    out_ref[...] = jnp.dot(hid, wb_ref[...],
                           preferred_element_type=jnp.float32) + bb_ref[...]


_headk = pl.pallas_call(
    _head_body,
    out_shape=jax.ShapeDtypeStruct((NG, 1), jnp.float32),
)


def kernel(x, edge_index, batch, molecule_features, W1, b1, W2, b2,
           Wm, bm, gamma, beta, Wa, ba, Wb, bb):
    src = edge_index[0]
    dst = edge_index[1]
    padi = jnp.full((EPAD - E,), N, jnp.int32)
    n0 = 16 * CHT0 * K

    def _split(flat):
        fill = N | (N << 16)
        p0 = jnp.pad(flat[:n0].reshape(16, CHT0, K),
                     ((0, 0), (0, CHTM - CHT0), (0, 0)), constant_values=fill)
        p1 = jnp.pad(flat[n0:].reshape(16, CHT1, K),
                     ((0, 0), (0, CHTM - CHT1), (0, 0)), constant_values=fill)
        return jnp.stack([p0, p1], axis=1).reshape(NTILES, CHTM, K)

    pk = (jnp.concatenate([src, padi])
          | (jnp.concatenate([dst, padi]) << 16))
    pk2 = _split(pk).reshape(NTILES, CHTM * K)
    xpad = jnp.pad(x, ((0, NPAD - N), (0, 0)))
    zeros = jnp.zeros((NPAD, D), jnp.float32)

    degp = _deg_kernel(pk2)
    dinv, xs = _prep(degp.T, xpad)
    p1 = _conv_kernel(xs, pk2, zeros)
    x1, xs1 = _tail(p1, dinv, xpad, W1, b1.reshape(1, D))
    p2 = _conv_kernel(xs1, pk2, zeros)
    x2, _ = _tail(p2, dinv, x1, W2, b2.reshape(1, D))

    bp = jnp.full((NTILES, BPAD), NG, jnp.int32)
    bp = bp.at[:, :POOL_T].set(
        jnp.pad(batch, (0, NPAD - N), constant_values=NG).reshape(
            NTILES, POOL_T))
    mxp, smp = _pool_kernel(x2.reshape(NTILES, POOL_T * D), bp)
    out = _headk(mxp.reshape(NTILES * GB, D), smp.reshape(NTILES * GB, D),
                 molecule_features, Wm, bm.reshape(1, -1),
                 gamma.reshape(1, -1), beta.reshape(1, -1),
                 Wa, ba.reshape(1, D), Wb, bb.reshape(1, 1))
    return out


# speedup vs baseline: 1.1820x; 1.0570x over previous
import functools

import jax
import jax.numpy as jnp
from jax import lax
from jax.experimental import pallas as pl
from jax.experimental.pallas import tpu as pltpu
from jax.experimental.pallas import tpu_sc as plsc

N = 10000
NPAD = 10112
D = 128
E = 320000
K = 144
NTILES = 32
CHT0 = 70
CHT1 = 70
CHTM = max(CHT0, CHT1)
EPAD = 16 * (CHT0 + CHT1) * K
NG = 128
GB = 136
POOL_T = NPAD // NTILES
BPAD = 336

_mesh = plsc.VectorSubcoreMesh(core_axis_name="c", subcore_axis_name="s")
_sc_params = pltpu.CompilerParams(needs_layout_passes=False,
                                 use_tc_tiling_on_sc=False)


@functools.partial(
    pl.kernel, mesh=_mesh,
    out_type=jax.ShapeDtypeStruct((NTILES, NPAD), jnp.float32),
    scratch_types=[pltpu.VMEM((CHTM * K,), jnp.int32),
                   pltpu.VMEM((NPAD,), jnp.float32)],
    compiler_params=_sc_params,
)
def _deg_kernel(pk_hbm, out_hbm, idxv, accv):
    wid = lax.axis_index("s") * 2 + lax.axis_index("c")
    pltpu.sync_copy(pk_hbm.at[wid], idxv)
    zero16 = jnp.zeros((16,), jnp.float32)

    def _z(i, c):
        accv[pl.ds(i * 16, 16)] = zero16
        return c
    lax.fori_loop(0, NPAD // 16, _z, 0)

    ones16 = jnp.ones((16,), jnp.float32)

    def _s(i, c):
        d16 = lax.shift_right_logical(idxv[pl.ds(i * 16, 16)], 16)
        plsc.addupdate_scatter(accv, [d16], ones16)
        return c
    lax.fori_loop(0, CHTM * K // 16, _s, 0)
    pltpu.sync_copy(accv, out_hbm.at[wid])


@functools.partial(
    pl.kernel, mesh=_mesh,
    out_type=jax.ShapeDtypeStruct((2, NPAD, D), jnp.float32),
    scratch_types=[pltpu.VMEM((CHTM * K,), jnp.int32),
                   pltpu.VMEM((K,), jnp.int32),
                   pltpu.VMEM((K,), jnp.int32),
                   pltpu.VMEM((K,), jnp.int32),
                   pltpu.VMEM((K,), jnp.int32),
                   pltpu.VMEM((K, D), jnp.float32),
                   pltpu.VMEM((K, D), jnp.float32),
                   pltpu.VMEM_SHARED((NPAD, D), jnp.float32),
                   pltpu.SemaphoreType.DMA,
                   pltpu.SemaphoreType.DMA],
    compiler_params=_sc_params,
)
def _conv_kernel(xs_hbm, pk_hbm, zeros_hbm, out_hbm,
                 pkv, srcxa, dstxa, srcxb, dstxb, bufa, bufb, acc,
                 sema, semb):
    cid = lax.axis_index("c")
    sid = lax.axis_index("s")
    wid = sid * 2 + cid
    pltpu.sync_copy(pk_hbm.at[wid], pkv)
    nrows = NPAD // 16
    r0 = sid * nrows
    pltpu.sync_copy(zeros_hbm.at[pl.ds(r0, nrows)], acc.at[pl.ds(r0, nrows)])
    plsc.subcore_barrier()

    ncht = jnp.where(cid == 0, CHT0, CHT1)

    def _unpack(j, sx, dx):
        base = j * K
        for i in range(K // 16):
            p = pkv[pl.ds(base + i * 16, 16)]
            sx[pl.ds(i * 16, 16)] = p & 0xFFFF
            dx[pl.ds(i * 16, 16)] = lax.shift_right_logical(p, 16)

    _unpack(0, srcxa, dstxa)
    pltpu.async_copy(xs_hbm.at[srcxa], bufa, sema)

    def _pair(jj, c):
        j0 = jj * 2
        _unpack(j0 + 1, srcxb, dstxb)
        pltpu.make_async_copy(xs_hbm.at[srcxa], bufa, sema).wait()
        pltpu.async_copy(xs_hbm.at[srcxb], bufb, semb)
        pltpu.sync_copy(bufa, acc.at[dstxa], add=True)

        @pl.when(j0 + 2 < ncht)
        def _():
            _unpack(j0 + 2, srcxa, dstxa)
            pltpu.async_copy(xs_hbm.at[srcxa], bufa, sema)
        pltpu.make_async_copy(xs_hbm.at[srcxb], bufb, semb).wait()
        pltpu.sync_copy(bufb, acc.at[dstxb], add=True)
        return c
    lax.fori_loop(0, ncht // 2, _pair, 0)
    plsc.subcore_barrier()
    pltpu.sync_copy(acc.at[pl.ds(r0, nrows)],
                    out_hbm.at[cid, pl.ds(r0, nrows)])


@functools.partial(
    pl.kernel, mesh=_mesh,
    out_type=[jax.ShapeDtypeStruct((NTILES, GB * D), jnp.float32),
              jax.ShapeDtypeStruct((NTILES, GB * D), jnp.float32)],
    scratch_types=[pltpu.VMEM((POOL_T * D,), jnp.float32),
                   pltpu.VMEM((BPAD,), jnp.int32),
                   pltpu.VMEM((GB * D,), jnp.float32),
                   pltpu.VMEM((GB * D,), jnp.float32)],
    compiler_params=_sc_params,
)
def _pool_kernel(x2_hbm, batch_hbm, outmax_hbm, outsum_hbm, xv, bv, mxb, smb):
    wid = lax.axis_index("s") * 2 + lax.axis_index("c")
    pltpu.sync_copy(x2_hbm.at[wid], xv)
    pltpu.sync_copy(batch_hbm.at[wid], bv)
    neg16 = jnp.full((16,), -jnp.inf, jnp.float32)
    zero16 = jnp.zeros((16,), jnp.float32)

    def _z(i, c):
        mxb[pl.ds(i * 16, 16)] = neg16
        smb[pl.ds(i * 16, 16)] = zero16
        return c
    lax.fori_loop(0, GB * D // 16, _z, 0)

    def _n(i, c):
        g = bv[pl.ds(i, 16)][0]
        off = g * D
        xoff = i * D
        for j in range(D // 16):
            v = xv[pl.ds(xoff + j * 16, 16)]
            mxb[pl.ds(off + j * 16, 16)] = jnp.maximum(
                mxb[pl.ds(off + j * 16, 16)], v)
            smb[pl.ds(off + j * 16, 16)] = smb[pl.ds(off + j * 16, 16)] + v
        return c
    lax.fori_loop(0, POOL_T, _n, 0)
    pltpu.sync_copy(mxb, outmax_hbm.at[wid])
    pltpu.sync_copy(smb, outsum_hbm.at[wid])


def _prep_body(degp_ref, x_ref, dinv_ref, xs_ref):
    deg = jnp.sum(degp_ref[...], axis=1, keepdims=True) + 1.0
    dinv = lax.rsqrt(deg)
    mask = lax.broadcasted_iota(jnp.int32, (NPAD, 1), 0) < N
    dinv = jnp.where(mask, dinv, 0.0)
    dinv_ref[...] = dinv
    xs_ref[...] = x_ref[...] * dinv


_prep = pl.pallas_call(
    _prep_body,
    out_shape=[jax.ShapeDtypeStruct((NPAD, 1), jnp.float32),
               jax.ShapeDtypeStruct((NPAD, D), jnp.float32)],
)


def _tail_body(p_ref, dinv_ref, xin_ref, w_ref, b_ref, xn_ref, xsn_ref):
    t = p_ref[0] + p_ref[1]
    dinv = dinv_ref[...]
    agg = dinv * t + (dinv * dinv) * xin_ref[...]
    h = jnp.maximum(
        jnp.dot(agg, w_ref[...], preferred_element_type=jnp.float32)
        + b_ref[...], 0.0)
    xn = h + xin_ref[...]
    xn_ref[...] = xn
    xsn_ref[...] = xn * dinv


_tail = pl.pallas_call(
    _tail_body,
    out_shape=[jax.ShapeDtypeStruct((NPAD, D), jnp.float32),
               jax.ShapeDtypeStruct((NPAD, D), jnp.float32)],
)


def _head_body(mx_ref, sm_ref, mol_ref, wm_ref, bm_ref, g_ref, be_ref,
               wa_ref, ba_ref, wb_ref, bb_ref, out_ref):
    mx = jnp.full((GB, D), -jnp.inf, jnp.float32)
    sm = jnp.zeros((GB, D), jnp.float32)
    for w in range(NTILES):
        mx = jnp.maximum(mx, mx_ref[pl.ds(w * GB, GB), :])
        sm = sm + sm_ref[pl.ds(w * GB, GB), :]
    mx = mx[:NG]
    sm = sm[:NG]
    mol = jnp.dot(mol_ref[...], wm_ref[...],
                  preferred_element_type=jnp.float32) + bm_ref[...]
    mol = jnp.maximum(g_ref[...] * mol + be_ref[...], 0.0)
    cat = jnp.concatenate([mx, sm, mol], axis=1)
    hid = jnp.maximum(
        jnp.dot(cat, wa_ref[...], preferred_element_type=jnp.float32)
        + ba_ref[...], 0.0)
    out_ref[...] = jnp.dot(hid, wb_ref[...],
                           preferred_element_type=jnp.float32) + bb_ref[...]


_headk = pl.pallas_call(
    _head_body,
    out_shape=jax.ShapeDtypeStruct((NG, 1), jnp.float32),
)


def kernel(x, edge_index, batch, molecule_features, W1, b1, W2, b2,
           Wm, bm, gamma, beta, Wa, ba, Wb, bb):
    src = edge_index[0]
    dst = edge_index[1]
    padi = jnp.full((EPAD - E,), N, jnp.int32)
    n0 = 16 * CHT0 * K

    def _split(flat):
        fill = N | (N << 16)
        p0 = jnp.pad(flat[:n0].reshape(16, CHT0, K),
                     ((0, 0), (0, CHTM - CHT0), (0, 0)), constant_values=fill)
        p1 = jnp.pad(flat[n0:].reshape(16, CHT1, K),
                     ((0, 0), (0, CHTM - CHT1), (0, 0)), constant_values=fill)
        return jnp.stack([p0, p1], axis=1).reshape(NTILES, CHTM, K)

    pk = (jnp.concatenate([src, padi])
          | (jnp.concatenate([dst, padi]) << 16))
    pk2 = _split(pk).reshape(NTILES, CHTM * K)
    xpad = jnp.pad(x, ((0, NPAD - N), (0, 0)))
    zeros = jnp.zeros((NPAD, D), jnp.float32)

    degp = _deg_kernel(pk2)
    dinv, xs = _prep(degp.T, xpad)
    p1 = _conv_kernel(xs, pk2, zeros)
    x1, xs1 = _tail(p1, dinv, xpad, W1, b1.reshape(1, D))
    p2 = _conv_kernel(xs1, pk2, zeros)
    x2, _ = _tail(p2, dinv, x1, W2, b2.reshape(1, D))

    bp = jnp.full((NTILES, BPAD), NG, jnp.int32)
    bp = bp.at[:, :POOL_T].set(
        jnp.pad(batch, (0, NPAD - N), constant_values=NG).reshape(
            NTILES, POOL_T))
    mxp, smp = _pool_kernel(x2.reshape(NTILES, POOL_T * D), bp)
    out = _headk(mxp.reshape(NTILES * GB, D), smp.reshape(NTILES * GB, D),
                 molecule_features, Wm, bm.reshape(1, -1),
                 gamma.reshape(1, -1), beta.reshape(1, -1),
                 Wa, ba.reshape(1, D), Wb, bb.reshape(1, 1))
    return out
